# baseline (speedup 1.0000x reference)
import jax
import jax.numpy as jnp
from jax import lax
from jax.experimental import pallas as pl
from jax.experimental.pallas import tpu as pltpu
from jax.experimental.pallas import tpu_sc as plsc

_RES = 256
_L = 16
_NC, _NS = 2, 16
_NW = _NC * _NS
_B = 128
_P = 4096
_NBUF = 2


def _compute_chunk(in_v, tabs, out_v):
    siota = lax.iota(jnp.int32, _L) * 6
    @plsc.parallel_loop(0, _P // _B, unroll=2)
    def _(gi):
        ib = gi * (4 * _B)
        ob = gi * (6 * _B)
        for k in range(_B // _L):
            o = k * _L
            w = in_v[pl.ds(ib + 3 * _B + o, _L)]
            idx = jnp.clip(w.astype(jnp.int32), 0, _RES - 1)
            for c in range(3):
                plsc.store_scatter(
                    out_v, [siota + (ob + o * 6 + c)],
                    in_v[pl.ds(ib + c * _B + o, _L)])
                plsc.store_scatter(
                    out_v, [siota + (ob + o * 6 + 3 + c)],
                    plsc.load_gather(tabs[c], [idx]))


def _body(points_hbm, colors_hbm, out_hbm, tab0, tab1, tab2,
          in_v0, in_v1, out_v0, out_v1, sin0, sin1, sout0, sout1):
    ins, outs = [in_v0, in_v1], [out_v0, out_v1]
    sins, souts = [sin0, sin1], [sout0, sout1]
    tabs = [tab0, tab1, tab2]
    wid = lax.axis_index("s") * _NC + lax.axis_index("c")
    per_tile = points_hbm.shape[0] // (4 * _NW)
    chunks = per_tile // _P
    tile_base = wid * per_tile

    for c in range(3):
        pltpu.sync_copy(colors_hbm.at[pl.ds(c * _RES, _RES)], tabs[c])
    for b in range(_NBUF):
        pltpu.async_copy(
            points_hbm.at[pl.ds((tile_base + b * _P) * 4, _P * 4)],
            ins[b], sins[b])
    for c in range(3):
        for j in range(_RES // _L):
            tabs[c][pl.ds(j * _L, _L)] = \
                jnp.clip(tabs[c][pl.ds(j * _L, _L)], 0.0, 1.0)

    def outer(g, carry):
        for b in range(_NBUF):
            t = g * _NBUF + b
            base = tile_base + t * _P
            pltpu.make_async_copy(
                points_hbm.at[pl.ds(base * 4, _P * 4)], ins[b], sins[b]).wait()

            @pl.when(t >= _NBUF)
            def _():
                prev = tile_base + (t - _NBUF) * _P
                pltpu.make_async_copy(
                    outs[b], out_hbm.at[pl.ds(prev * 6, _P * 6)],
                    souts[b]).wait()

            _compute_chunk(ins[b], tabs, outs[b])
            pltpu.async_copy(
                outs[b], out_hbm.at[pl.ds(base * 6, _P * 6)], souts[b])

            @pl.when(t + _NBUF < chunks)
            def _():
                nxt = tile_base + (t + _NBUF) * _P
                pltpu.async_copy(
                    points_hbm.at[pl.ds(nxt * 4, _P * 4)], ins[b], sins[b])
        return carry

    lax.fori_loop(0, chunks // _NBUF, outer, 0)
    for b in range(_NBUF):
        base = tile_base + (chunks - _NBUF + b) * _P
        pltpu.make_async_copy(
            outs[b], out_hbm.at[pl.ds(base * 6, _P * 6)], souts[b]).wait()


def kernel(points, colors):
    n = points.shape[0]
    g = n // _B
    f = pl.kernel(
        _body,
        out_type=jax.ShapeDtypeStruct((n * 6,), jnp.float32),
        mesh=plsc.VectorSubcoreMesh(
            core_axis_name="c", subcore_axis_name="s",
            num_cores=_NC, num_subcores=_NS),
        scratch_types=[
            pltpu.VMEM((_RES,), jnp.float32),
            pltpu.VMEM((_RES,), jnp.float32),
            pltpu.VMEM((_RES,), jnp.float32),
            pltpu.VMEM((_P * 4,), jnp.float32),
            pltpu.VMEM((_P * 4,), jnp.float32),
            pltpu.VMEM((_P * 6,), jnp.float32),
            pltpu.VMEM((_P * 6,), jnp.float32),
            pltpu.SemaphoreType.DMA,
            pltpu.SemaphoreType.DMA,
            pltpu.SemaphoreType.DMA,
            pltpu.SemaphoreType.DMA,
        ],
        compiler_params=pltpu.CompilerParams(
            needs_layout_passes=False, use_tc_tiling_on_sc=False),
    )
    pts_soa = points.reshape(g, _B, 4).transpose(0, 2, 1).reshape(n * 4)
    colors_t = colors.T.reshape(3 * _RES)
    out6 = f(pts_soa, colors_t)
    return out6.reshape(n, 6)


# speedup vs baseline: 16.5736x; 16.5736x over previous
import jax
import jax.numpy as jnp
from jax import lax
from jax.experimental import pallas as pl
from jax.experimental.pallas import tpu as pltpu
from jax.experimental.pallas import tpu_sc as plsc

_RES = 256
_L = 16
_NC, _NS = 2, 16
_NW = _NC * _NS
_B = 128
_P = 4096
_NBUF = 2


def _compute_chunk(in_v, tabs, out_v):
    @plsc.parallel_loop(0, _P // _B, unroll=2)
    def _(gi):
        ib = gi * (4 * _B)
        for k in range(_B // _L):
            o = k * _L
            w = in_v[pl.ds(ib + 3 * _B + o, _L)]
            idx = jnp.clip(w.astype(jnp.int32), 0, _RES - 1)
            for c in range(3):
                out_v[gi, c, pl.ds(o, _L)] = \
                    in_v[pl.ds(ib + c * _B + o, _L)]
                out_v[gi, 3 + c, pl.ds(o, _L)] = \
                    plsc.load_gather(tabs[c], [idx])


def _body(points_hbm, colors_hbm, out_hbm, tab0, tab1, tab2,
          in_v0, in_v1, out_v0, out_v1, sin0, sin1, sout0, sout1):
    ins, outs = [in_v0, in_v1], [out_v0, out_v1]
    sins, souts = [sin0, sin1], [sout0, sout1]
    tabs = [tab0, tab1, tab2]
    wid = lax.axis_index("s") * _NC + lax.axis_index("c")
    per_tile = points_hbm.shape[0] // (4 * _NW)
    chunks = per_tile // _P
    tile_base = wid * per_tile

    for c in range(3):
        pltpu.sync_copy(colors_hbm.at[pl.ds(c * _RES, _RES)], tabs[c])
    for b in range(_NBUF):
        pltpu.async_copy(
            points_hbm.at[pl.ds((tile_base + b * _P) * 4, _P * 4)],
            ins[b], sins[b])
    for c in range(3):
        for j in range(_RES // _L):
            tabs[c][pl.ds(j * _L, _L)] = \
                jnp.clip(tabs[c][pl.ds(j * _L, _L)], 0.0, 1.0)

    def outer(g, carry):
        for b in range(_NBUF):
            t = g * _NBUF + b
            base = tile_base + t * _P
            pltpu.make_async_copy(
                points_hbm.at[pl.ds(base * 4, _P * 4)], ins[b], sins[b]).wait()

            @pl.when(t >= _NBUF)
            def _():
                prev = tile_base + (t - _NBUF) * _P
                pltpu.make_async_copy(
                    outs[b].at[:, pl.ds(0, 6)],
                    out_hbm.at[pl.ds(prev // _B, _P // _B), pl.ds(0, 6)],
                    souts[b]).wait()

            _compute_chunk(ins[b], tabs, outs[b])
            pltpu.async_copy(
                outs[b].at[:, pl.ds(0, 6)],
                out_hbm.at[pl.ds(base // _B, _P // _B), pl.ds(0, 6)],
                souts[b])

            @pl.when(t + _NBUF < chunks)
            def _():
                nxt = tile_base + (t + _NBUF) * _P
                pltpu.async_copy(
                    points_hbm.at[pl.ds(nxt * 4, _P * 4)], ins[b], sins[b])
        return carry

    lax.fori_loop(0, chunks // _NBUF, outer, 0)
    for b in range(_NBUF):
        base = tile_base + (chunks - _NBUF + b) * _P
        pltpu.make_async_copy(
            outs[b].at[:, pl.ds(0, 6)],
            out_hbm.at[pl.ds(base // _B, _P // _B), pl.ds(0, 6)],
            souts[b]).wait()


def kernel(points, colors):
    n = points.shape[0]
    g = n // _B
    f = pl.kernel(
        _body,
        out_type=jax.ShapeDtypeStruct((g, 8, _B), jnp.float32),
        mesh=plsc.VectorSubcoreMesh(
            core_axis_name="c", subcore_axis_name="s",
            num_cores=_NC, num_subcores=_NS),
        scratch_types=[
            pltpu.VMEM((_RES,), jnp.float32),
            pltpu.VMEM((_RES,), jnp.float32),
            pltpu.VMEM((_RES,), jnp.float32),
            pltpu.VMEM((_P * 4,), jnp.float32),
            pltpu.VMEM((_P * 4,), jnp.float32),
            pltpu.VMEM((_P // _B, 8, _B), jnp.float32),
            pltpu.VMEM((_P // _B, 8, _B), jnp.float32),
            pltpu.SemaphoreType.DMA,
            pltpu.SemaphoreType.DMA,
            pltpu.SemaphoreType.DMA,
            pltpu.SemaphoreType.DMA,
        ],
        compiler_params=pltpu.CompilerParams(
            needs_layout_passes=False, use_tc_tiling_on_sc=False),
    )
    pts_soa = points.reshape(g, _B, 4).transpose(0, 2, 1).reshape(n * 4)
    colors_t = colors.T.reshape(3 * _RES)
    out8 = f(pts_soa, colors_t)
    return out8[:, :6, :].transpose(0, 2, 1).reshape(n, 6)
